# Initial kernel scaffold; baseline (speedup 1.0000x reference)
import jax
import jax.numpy as jnp
from jax.experimental import pallas as pl


def kernel(h, x, W_e1, b_e1, W_e2, b_e2, W_h1, b_h1, W_h2, b_h2, W_c, b_c, W_s, b_s):
    raise NotImplementedError("write your pallas kernel here")


# speedup vs baseline: 13.7977x; 13.7977x over previous
import functools

import jax
import jax.numpy as jnp
from jax import lax
from jax.experimental import pallas as pl

_K = 20
_BLK = 256


def _body(h_ref, xb_ref, xa_ref, xt_ref, we1_ref, be1_ref, we2_ref, be2_ref,
          wh1_ref, bh1_ref, wh2_ref, bh2_ref, wct_ref, bc_ref, wst_ref, bs_ref,
          hout_ref, xout_ref, *, n, d_h, d_x, hid, blk):
    i = pl.program_id(1)

    h_all = h_ref[...]
    h_blk = h_ref[pl.ds(i * blk, blk), :]
    x_blk = xb_ref[...]
    x_all = xa_ref[...]
    xt = xt_ref[...]

    d = jnp.zeros((blk, n), jnp.float32)
    for c in range(d_x):
        diff = x_blk[:, c:c + 1] - xt[c:c + 1, :]
        d = d + diff * diff
    row_ids = i * blk + lax.broadcasted_iota(jnp.int32, (blk, 1), 0)
    col_ids = lax.broadcasted_iota(jnp.int32, (blk, n), 1)
    d = jnp.where(col_ids == row_ids, d + 1e10, d)

    g_all = jnp.dot(h_all, we1_ref[pl.ds(d_h, d_h), :],
                    preferred_element_type=jnp.float32)
    h1_blk = jnp.dot(h_blk, we1_ref[pl.ds(0, d_h), :],
                     preferred_element_type=jnp.float32)
    w_last = we1_ref[pl.ds(2 * d_h, 1), :]
    be1 = be1_ref[...]
    be2 = be2_ref[...]
    wct = wct_ref[...]
    bc = bc_ref[...]
    we2 = we2_ref[...]

    m_acc = jnp.zeros((blk, hid), jnp.float32)
    x_acc = jnp.zeros((blk, d_x), jnp.float32)

    for _ in range(_K):
        d2 = jnp.min(d, axis=1, keepdims=True)
        is_min = d == d2
        idx = jnp.min(jnp.where(is_min, col_ids, n), axis=1, keepdims=True)
        onehot = (col_ids == idx).astype(jnp.float32)
        g_j = jnp.dot(onehot, g_all, preferred_element_type=jnp.float32)
        x_j = jnp.dot(onehot, x_all, preferred_element_type=jnp.float32)
        e = h1_blk + g_j + d2 * w_last + be1
        s = e * jax.nn.sigmoid(e)
        m_ij = jnp.dot(s, we2, preferred_element_type=jnp.float32) + be2
        m_acc = m_acc + m_ij
        w = jnp.sum(m_ij * wct, axis=1, keepdims=True) + bc
        x_acc = x_acc + w * (x_j - x_blk)
        d = jnp.where(col_ids == idx, 1e30, d)

    t = (jnp.dot(h_blk, wh1_ref[pl.ds(0, d_h), :],
                 preferred_element_type=jnp.float32)
         + jnp.dot(m_acc, wh1_ref[pl.ds(d_h, hid), :],
                   preferred_element_type=jnp.float32)
         + bh1_ref[...])
    t = t * jax.nn.sigmoid(t)
    h_new = (jnp.dot(t, wh2_ref[...], preferred_element_type=jnp.float32)
             + bh2_ref[...] + h_blk)
    hout_ref[...] = h_new

    scale = jnp.tanh(jnp.sum(h_new * wst_ref[...], axis=1, keepdims=True)
                     + bs_ref[...])
    norm = jnp.sqrt(jnp.sum(x_acc * x_acc, axis=1, keepdims=True)) + 1e-8
    xout_ref[...] = x_blk + scale * (x_acc / norm) * 0.1


def kernel(h, x, W_e1, b_e1, W_e2, b_e2, W_h1, b_h1, W_h2, b_h2,
           W_c, b_c, W_s, b_s):
    b_sz, n, d_h = h.shape
    d_x = x.shape[-1]
    hid = W_e2.shape[0]
    blk = _BLK
    while n % blk:
        blk //= 2
    grid = (b_sz, n // blk)

    xt = jnp.swapaxes(x, 1, 2)
    be1 = b_e1.reshape(1, hid)
    be2 = b_e2.reshape(1, hid)
    bh1 = b_h1.reshape(1, d_h)
    bh2 = b_h2.reshape(1, d_h)
    wct = W_c.reshape(1, hid)
    bc = b_c.reshape(1, 1)
    wst = W_s.reshape(1, d_h)
    bs = b_s.reshape(1, 1)

    full = lambda shape: pl.BlockSpec(shape, lambda b, i: (0,) * len(shape))
    in_specs = [
        pl.BlockSpec((None, n, d_h), lambda b, i: (b, 0, 0)),
        pl.BlockSpec((None, blk, d_x), lambda b, i: (b, i, 0)),
        pl.BlockSpec((None, n, d_x), lambda b, i: (b, 0, 0)),
        pl.BlockSpec((None, d_x, n), lambda b, i: (b, 0, 0)),
        full((2 * d_h + 1, hid)), full((1, hid)),
        full((hid, hid)), full((1, hid)),
        full((d_h + hid, d_h)), full((1, d_h)),
        full((d_h, d_h)), full((1, d_h)),
        full((1, hid)), full((1, 1)),
        full((1, d_h)), full((1, 1)),
    ]
    out_specs = [
        pl.BlockSpec((None, blk, d_h), lambda b, i: (b, i, 0)),
        pl.BlockSpec((None, blk, d_x), lambda b, i: (b, i, 0)),
    ]
    out_shape = [
        jax.ShapeDtypeStruct((b_sz, n, d_h), jnp.float32),
        jax.ShapeDtypeStruct((b_sz, n, d_x), jnp.float32),
    ]

    body = functools.partial(_body, n=n, d_h=d_h, d_x=d_x, hid=hid, blk=blk)
    h_new, x_new = pl.pallas_call(
        body,
        grid=grid,
        in_specs=in_specs,
        out_specs=out_specs,
        out_shape=out_shape,
    )(h, x, x, xt, W_e1, be1, W_e2, be2, W_h1, bh1, W_h2, bh2,
      wct, bc, wst, bs)
    return h_new, x_new
